# baseline (device time: 32000 ns/iter reference)
import jax
import jax.numpy as jnp
from jax import lax
from jax.experimental import pallas as pl
from jax.experimental.pallas import tpu as pltpu

N_DEV = 32


def kernel(table, idx):
    rows_per, d = table.shape
    n = idx.shape[0]
    chunk = n // N_DEV

    def body(table_ref, idx_ref, out_ref, partial, rs_buf,
             rs_send, rs_recv, ag_send, ag_recv):
        my = lax.axis_index("i")

        barrier = pltpu.get_barrier_semaphore()
        for o in range(1, N_DEV):
            pl.semaphore_signal(
                barrier, inc=1,
                device_id=(my ^ o,), device_id_type=pl.DeviceIdType.MESH,
            )
        pl.semaphore_wait(barrier, N_DEV - 1)

        ids = idx_ref[...]
        li = ids - my * rows_per
        in_range = (li >= 0) & (li < rows_per)
        col = lax.broadcasted_iota(jnp.int32, (n, rows_per), 1)
        oh = ((col == li) & in_range).astype(jnp.float32)
        partial[...] = jnp.dot(
            oh, table_ref[...], preferred_element_type=jnp.float32
        )

        rs = []
        for o in range(1, N_DEV):
            peer = my ^ o
            r = pltpu.make_async_remote_copy(
                src_ref=partial.at[pl.ds(peer * chunk, chunk)],
                dst_ref=rs_buf.at[o],
                send_sem=rs_send.at[o],
                recv_sem=rs_recv.at[o],
                device_id=(peer,),
                device_id_type=pl.DeviceIdType.MESH,
            )
            r.start()
            rs.append(r)

        acc = partial[pl.ds(my * chunk, chunk), :]
        for o in range(1, N_DEV):
            rs[o - 1].wait_recv()
            acc = acc + rs_buf[o]
        out_ref[pl.ds(my * chunk, chunk), :] = acc

        ag = []
        for o in range(1, N_DEV):
            peer = my ^ o
            r = pltpu.make_async_remote_copy(
                src_ref=out_ref.at[pl.ds(my * chunk, chunk)],
                dst_ref=out_ref.at[pl.ds(my * chunk, chunk)],
                send_sem=ag_send.at[o],
                recv_sem=ag_recv.at[o],
                device_id=(peer,),
                device_id_type=pl.DeviceIdType.MESH,
            )
            r.start()
            ag.append(r)

        for o in range(1, N_DEV):
            peer = my ^ o
            w = pltpu.make_async_remote_copy(
                src_ref=out_ref.at[pl.ds(peer * chunk, chunk)],
                dst_ref=out_ref.at[pl.ds(peer * chunk, chunk)],
                send_sem=ag_send.at[o],
                recv_sem=ag_recv.at[o],
                device_id=(peer,),
                device_id_type=pl.DeviceIdType.MESH,
            )
            w.wait_recv()

        for r in rs:
            r.wait_send()
        for r in ag:
            r.wait_send()

    return pl.pallas_call(
        body,
        out_shape=jax.ShapeDtypeStruct((n, d), jnp.float32),
        in_specs=[
            pl.BlockSpec(memory_space=pltpu.VMEM),
            pl.BlockSpec(memory_space=pltpu.VMEM),
        ],
        out_specs=pl.BlockSpec(memory_space=pltpu.VMEM),
        scratch_shapes=[
            pltpu.VMEM((n, d), jnp.float32),
            pltpu.VMEM((N_DEV, chunk, d), jnp.float32),
            pltpu.SemaphoreType.DMA((N_DEV,)),
            pltpu.SemaphoreType.DMA((N_DEV,)),
            pltpu.SemaphoreType.DMA((N_DEV,)),
            pltpu.SemaphoreType.DMA((N_DEV,)),
        ],
        compiler_params=pltpu.CompilerParams(collective_id=0),
    )(table, idx.reshape(n, 1))


# device time: 28156 ns/iter; 1.1365x vs baseline; 1.1365x over previous
import jax
import jax.numpy as jnp
from jax import lax
from jax.experimental import pallas as pl
from jax.experimental.pallas import tpu as pltpu

N_DEV = 32


def kernel(table, idx):
    rows_per, d = table.shape
    n = idx.shape[0]
    chunk = n // N_DEV

    def body(table_ref, idx_ref, out_ref, partial, rs_buf,
             rs_send, rs_recv, ag_send, ag_recv):
        my = lax.axis_index("i")

        barrier = pltpu.get_barrier_semaphore()
        for o in range(1, N_DEV):
            pl.semaphore_signal(
                barrier, inc=1,
                device_id=(my ^ o,), device_id_type=pl.DeviceIdType.MESH,
            )

        ids = idx_ref[...]
        li = ids - my * rows_per
        in_range = (li >= 0) & (li < rows_per)
        col = lax.broadcasted_iota(jnp.int32, (n, rows_per), 1)
        oh = ((col == li) & in_range).astype(jnp.float32)
        partial[...] = jnp.dot(
            oh, table_ref[...], preferred_element_type=jnp.float32
        )

        pl.semaphore_wait(barrier, N_DEV - 1)

        rs = []
        for o in range(N_DEV - 1, 0, -1):
            peer = my ^ o
            r = pltpu.make_async_remote_copy(
                src_ref=partial.at[pl.ds(peer * chunk, chunk)],
                dst_ref=rs_buf.at[o],
                send_sem=rs_send.at[o],
                recv_sem=rs_recv.at[o],
                device_id=(peer,),
                device_id_type=pl.DeviceIdType.MESH,
            )
            r.start()
            rs.append(r)

        for r in rs:
            r.wait_recv()
        rs_buf[0, :, :] = partial[pl.ds(my * chunk, chunk), :]
        vals = [rs_buf[o] for o in range(N_DEV)]
        while len(vals) > 1:
            vals = [
                vals[i] + vals[i + 1] if i + 1 < len(vals) else vals[i]
                for i in range(0, len(vals), 2)
            ]
        out_ref[pl.ds(my * chunk, chunk), :] = vals[0]

        ag = []
        for o in range(N_DEV - 1, 0, -1):
            peer = my ^ o
            r = pltpu.make_async_remote_copy(
                src_ref=out_ref.at[pl.ds(my * chunk, chunk)],
                dst_ref=out_ref.at[pl.ds(my * chunk, chunk)],
                send_sem=ag_send.at[o],
                recv_sem=ag_recv.at[o],
                device_id=(peer,),
                device_id_type=pl.DeviceIdType.MESH,
            )
            r.start()
            ag.append(r)

        for o in range(1, N_DEV):
            peer = my ^ o
            w = pltpu.make_async_remote_copy(
                src_ref=out_ref.at[pl.ds(peer * chunk, chunk)],
                dst_ref=out_ref.at[pl.ds(peer * chunk, chunk)],
                send_sem=ag_send.at[o],
                recv_sem=ag_recv.at[o],
                device_id=(peer,),
                device_id_type=pl.DeviceIdType.MESH,
            )
            w.wait_recv()

        for r in rs:
            r.wait_send()
        for r in ag:
            r.wait_send()

    return pl.pallas_call(
        body,
        out_shape=jax.ShapeDtypeStruct((n, d), jnp.float32),
        in_specs=[
            pl.BlockSpec(memory_space=pltpu.VMEM),
            pl.BlockSpec(memory_space=pltpu.VMEM),
        ],
        out_specs=pl.BlockSpec(memory_space=pltpu.VMEM),
        scratch_shapes=[
            pltpu.VMEM((n, d), jnp.float32),
            pltpu.VMEM((N_DEV, chunk, d), jnp.float32),
            pltpu.SemaphoreType.DMA((N_DEV,)),
            pltpu.SemaphoreType.DMA((N_DEV,)),
            pltpu.SemaphoreType.DMA((N_DEV,)),
            pltpu.SemaphoreType.DMA((N_DEV,)),
        ],
        compiler_params=pltpu.CompilerParams(collective_id=0),
    )(table, idx.reshape(n, 1))
